# baseline (speedup 1.0000x reference)
import functools

import jax
import jax.numpy as jnp
from jax import lax
from jax.experimental import pallas as pl
from jax.experimental.pallas import tpu as pltpu
from jax.experimental.pallas import tpu_sc as plsc

_B, _T, _EMB = 4096, 200, 64
_VOCAB = 1000000
_NPAIR = _VOCAB // 2
_NW = 32
_BBLK = _B // _NW
_L = 16
_NJ = _BBLK // _L

_VW = 256
_NFULL = _VOCAB // _VW
_TAILV = _VOCAB - _NFULL * _VW
_CPW = _NFULL // _NW

_mesh = plsc.VectorSubcoreMesh(core_axis_name="c", subcore_axis_name="s")


@functools.partial(
    pl.kernel,
    out_type=jax.ShapeDtypeStruct((_NPAIR, 2 * _EMB), jnp.float32),
    mesh=_mesh,
    scratch_types=[
        pltpu.VMEM((_EMB, _VW), jnp.float32),
        pltpu.VMEM((_EMB, _VW), jnp.float32),
        pltpu.VMEM((_VW // 2, 2 * _EMB), jnp.float32),
        pltpu.VMEM((_VW // 2, 2 * _EMB), jnp.float32),
        pltpu.VMEM((_EMB, _TAILV), jnp.float32),
        pltpu.VMEM((_TAILV // 2, 2 * _EMB), jnp.float32),
        pltpu.SemaphoreType.DMA,
        pltpu.SemaphoreType.DMA,
        pltpu.SemaphoreType.DMA,
        pltpu.SemaphoreType.DMA,
    ],
    compiler_params=pltpu.CompilerParams(needs_layout_passes=False),
)
def _sc_transpose(tt_hbm, tp_hbm, ib0, ib1, ob0, ob1, tib, tob,
                  r0, r1, w0, w1):
    nc = plsc.get_sparse_core_info().num_cores
    wid = lax.axis_index("s") * nc + lax.axis_index("c")

    ibuf = (ib0, ib1)
    obuf = (ob0, ob1)
    rsem = (r0, r1)
    wsem = (w0, w1)

    def fire_read(cid, s):
        pltpu.async_copy(tt_hbm.at[:, pl.ds(cid * _VW, _VW)], ibuf[s],
                         rsem[s])

    def wait_read(s):
        pltpu.make_async_copy(tt_hbm.at[:, pl.ds(0, _VW)], ibuf[s],
                              rsem[s]).wait()

    def transpose_chunk(src, dst, npair):
        iot = lax.iota(jnp.int32, _L)
        for r0 in range(0, npair, _L // 2):
            cols_src = iot + (2 * r0)
            rows_dst = lax.shift_right_logical(iot, 1) + r0
            gsel = lax.shift_left(lax.bitwise_and(iot, 1), 6)

            @plsc.parallel_loop(0, _EMB, unroll=8)
            def eloop(e0):
                ev = lax.bitwise_and(iot + e0, _EMB - 1)
                v = plsc.load_gather(src, [ev, cols_src])
                plsc.store_scatter(dst, [rows_dst, gsel + ev], v)

    def fire_write(cid, s):
        pltpu.async_copy(obuf[s], tp_hbm.at[pl.ds(cid * (_VW // 2), _VW // 2)],
                         wsem[s])

    def wait_write(s):
        pltpu.make_async_copy(obuf[s], tp_hbm.at[pl.ds(0, _VW // 2)],
                              wsem[s]).wait()

    fire_read(wid, 0)
    fire_read(wid + _NW, 1)

    for s in range(2):
        wait_read(s)
        transpose_chunk(ibuf[s], obuf[s], _VW // 2)
        fire_write(wid + s * _NW, s)
        fire_read(wid + (s + 2) * _NW, s)

    def body(i, carry):
        for s in range(2):
            idx = 2 + 2 * i + s
            cid = wid + idx * _NW
            wait_read(s)
            wait_write(s)
            transpose_chunk(ibuf[s], obuf[s], _VW // 2)
            fire_write(cid, s)

            @pl.when(idx + 2 < _CPW)
            def _():
                fire_read(wid + (idx + 2) * _NW, s)
        return carry

    lax.fori_loop(0, (_CPW - 2) // 2, body, 0)
    for s in range(2):
        wait_write(s)

    @pl.when(wid < 2)
    def _():
        cid = _NFULL - 2 + wid
        pltpu.sync_copy(tt_hbm.at[:, pl.ds(cid * _VW, _VW)], ib0)
        transpose_chunk(ib0, ob0, _VW // 2)
        pltpu.sync_copy(ob0, tp_hbm.at[pl.ds(cid * (_VW // 2), _VW // 2)])

    @pl.when(wid == 2)
    def _():
        v0 = _NFULL * _VW
        pltpu.sync_copy(tt_hbm.at[:, pl.ds(v0, _TAILV)], tib)
        transpose_chunk(tib, tob, _TAILV // 2)
        pltpu.sync_copy(tob, tp_hbm.at[pl.ds(v0 // 2, _TAILV // 2)])


@functools.partial(
    pl.kernel,
    out_type=jax.ShapeDtypeStruct((_T, _EMB, _B), jnp.float32),
    mesh=_mesh,
    scratch_types=[
        pltpu.VMEM((_T, _BBLK), jnp.int32),
        pltpu.VMEM((_BBLK,), jnp.int32),
        pltpu.VMEM((_BBLK,), jnp.int32),
        pltpu.VMEM((_BBLK,), jnp.int32),
        pltpu.VMEM((_BBLK,), jnp.int32),
        pltpu.VMEM((_BBLK, 2 * _EMB), jnp.float32),
        pltpu.VMEM((_BBLK, 2 * _EMB), jnp.float32),
        pltpu.VMEM((_EMB, _BBLK), jnp.float32),
        pltpu.VMEM((_EMB, _BBLK), jnp.float32),
        pltpu.SemaphoreType.DMA,
        pltpu.SemaphoreType.DMA,
        pltpu.SemaphoreType.DMA,
        pltpu.SemaphoreType.DMA,
    ],
    compiler_params=pltpu.CompilerParams(needs_layout_passes=False),
)
def _sc_gather(x_hbm, tbl_hbm, out_hbm, idx_all, gi0, gi1, hf0, hf1,
               gb0, gb1, ob0, ob1, g0, g1, s0, s1):
    nc = plsc.get_sparse_core_info().num_cores
    wid = lax.axis_index("s") * nc + lax.axis_index("c")
    b0 = wid * _BBLK

    gidx = (gi0, gi1)
    half = (hf0, hf1)
    gbuf = (gb0, gb1)
    obuf = (ob0, ob1)
    gsem = (g0, g1)
    ssem = (s0, s1)

    pltpu.sync_copy(x_hbm.at[:, pl.ds(b0, _BBLK)], idx_all)

    def prep_and_fire(t, s):
        for j in range(_NJ):
            iv = idx_all[t, pl.ds(j * _L, _L)]
            gidx[s][pl.ds(j * _L, _L)] = lax.shift_right_logical(iv, 1)
            half[s][pl.ds(j * _L, _L)] = lax.shift_left(
                lax.bitwise_and(iv, 1), 6)
        pltpu.async_copy(tbl_hbm.at[gidx[s]], gbuf[s], gsem[s])

    def wait_gather(s):
        pltpu.make_async_copy(tbl_hbm.at[pl.ds(0, _BBLK)], gbuf[s],
                              gsem[s]).wait()

    def transpose_block(s):
        p0 = tuple(half[s][pl.ds(j * _L, _L)] for j in range(_NJ))

        @plsc.parallel_loop(0, _EMB, unroll=8, carry=p0)
        def erow(e0, p):
            ev = lax.bitwise_and(lax.iota(jnp.int32, _L) + e0, _EMB - 1)
            for j in range(_NJ):
                rows_b = lax.iota(jnp.int32, _L) + (j * _L)
                v = plsc.load_gather(gbuf[s], [rows_b, p[j] + ev])
                plsc.store_scatter(obuf[s], [ev, rows_b], v)
            return p

    def fire_store(t, s):
        pltpu.async_copy(obuf[s], out_hbm.at[t, :, pl.ds(b0, _BBLK)],
                         ssem[s])

    def wait_store(s):
        pltpu.make_async_copy(obuf[s], out_hbm.at[0, :, pl.ds(0, _BBLK)],
                              ssem[s]).wait()

    prep_and_fire(0, 0)
    prep_and_fire(1, 1)

    for s in range(2):
        wait_gather(s)
        transpose_block(s)
        fire_store(s, s)
        prep_and_fire(s + 2, s)

    def body(i, carry):
        for s in range(2):
            t = 2 + 2 * i + s
            wait_gather(s)
            wait_store(s)
            transpose_block(s)
            fire_store(t, s)
            prep_and_fire(t + 2, s)
        return carry

    lax.fori_loop(0, (_T - 4) // 2, body, 0)

    for s in range(2):
        t = _T - 2 + s
        wait_gather(s)
        wait_store(s)
        transpose_block(s)
        fire_store(t, s)
    for s in range(2):
        wait_store(s)


def kernel(x, table):
    tt = table.T
    tp = _sc_transpose(tt)
    out_k = _sc_gather(x.T, tp)
    return out_k.transpose(2, 0, 1)


# speedup vs baseline: 1.0123x; 1.0123x over previous
import functools

import jax
import jax.numpy as jnp
from jax import lax
from jax.experimental import pallas as pl
from jax.experimental.pallas import tpu as pltpu
from jax.experimental.pallas import tpu_sc as plsc

_B, _T, _EMB = 4096, 200, 64
_VOCAB = 1000000
_NPAIR = _VOCAB // 2
_NW = 32
_BBLK = _B // _NW
_L = 16
_NJ = _BBLK // _L

_VW = 256
_NFULL = _VOCAB // _VW
_TAILV = _VOCAB - _NFULL * _VW
_CPW = _NFULL // _NW

_mesh = plsc.VectorSubcoreMesh(core_axis_name="c", subcore_axis_name="s")


@functools.partial(
    pl.kernel,
    out_type=jax.ShapeDtypeStruct((_NPAIR, 2 * _EMB), jnp.float32),
    mesh=_mesh,
    scratch_types=[
        pltpu.VMEM((_EMB, _VW), jnp.float32),
        pltpu.VMEM((_EMB, _VW), jnp.float32),
        pltpu.VMEM((_VW // 2, 2 * _EMB), jnp.float32),
        pltpu.VMEM((_VW // 2, 2 * _EMB), jnp.float32),
        pltpu.VMEM((_EMB, _TAILV), jnp.float32),
        pltpu.VMEM((_TAILV // 2, 2 * _EMB), jnp.float32),
        pltpu.SemaphoreType.DMA,
        pltpu.SemaphoreType.DMA,
        pltpu.SemaphoreType.DMA,
        pltpu.SemaphoreType.DMA,
    ],
    compiler_params=pltpu.CompilerParams(needs_layout_passes=False),
)
def _sc_transpose(tt_hbm, tp_hbm, ib0, ib1, ob0, ob1, tib, tob,
                  r0, r1, w0, w1):
    nc = plsc.get_sparse_core_info().num_cores
    wid = lax.axis_index("s") * nc + lax.axis_index("c")

    ibuf = (ib0, ib1)
    obuf = (ob0, ob1)
    rsem = (r0, r1)
    wsem = (w0, w1)

    def fire_read(cid, s):
        pltpu.async_copy(tt_hbm.at[:, pl.ds(cid * _VW, _VW)], ibuf[s],
                         rsem[s])

    def wait_read(s):
        pltpu.make_async_copy(tt_hbm.at[:, pl.ds(0, _VW)], ibuf[s],
                              rsem[s]).wait()

    def transpose_chunk(src, dst, npair):
        iot = lax.iota(jnp.int32, _L)
        for r0 in range(0, npair, _L // 2):
            cols_src = iot + (2 * r0)
            rows_dst = lax.shift_right_logical(iot, 1) + r0
            gsel = lax.shift_left(lax.bitwise_and(iot, 1), 6)

            @plsc.parallel_loop(0, _EMB, unroll=4)
            def eloop(e0):
                ev = lax.bitwise_and(iot + e0, _EMB - 1)
                v = plsc.load_gather(src, [ev, cols_src])
                plsc.store_scatter(dst, [rows_dst, gsel + ev], v)

    def fire_write(cid, s):
        pltpu.async_copy(obuf[s], tp_hbm.at[pl.ds(cid * (_VW // 2), _VW // 2)],
                         wsem[s])

    def wait_write(s):
        pltpu.make_async_copy(obuf[s], tp_hbm.at[pl.ds(0, _VW // 2)],
                              wsem[s]).wait()

    fire_read(wid, 0)
    fire_read(wid + _NW, 1)

    for s in range(2):
        wait_read(s)
        transpose_chunk(ibuf[s], obuf[s], _VW // 2)
        fire_write(wid + s * _NW, s)
        fire_read(wid + (s + 2) * _NW, s)

    def body(i, carry):
        for s in range(2):
            idx = 2 + 2 * i + s
            cid = wid + idx * _NW
            wait_read(s)
            wait_write(s)
            transpose_chunk(ibuf[s], obuf[s], _VW // 2)
            fire_write(cid, s)

            @pl.when(idx + 2 < _CPW)
            def _():
                fire_read(wid + (idx + 2) * _NW, s)
        return carry

    lax.fori_loop(0, (_CPW - 2) // 2, body, 0)
    for s in range(2):
        wait_write(s)

    @pl.when(wid < 2)
    def _():
        cid = _NFULL - 2 + wid
        pltpu.sync_copy(tt_hbm.at[:, pl.ds(cid * _VW, _VW)], ib0)
        transpose_chunk(ib0, ob0, _VW // 2)
        pltpu.sync_copy(ob0, tp_hbm.at[pl.ds(cid * (_VW // 2), _VW // 2)])

    @pl.when(wid == 2)
    def _():
        v0 = _NFULL * _VW
        pltpu.sync_copy(tt_hbm.at[:, pl.ds(v0, _TAILV)], tib)
        transpose_chunk(tib, tob, _TAILV // 2)
        pltpu.sync_copy(tob, tp_hbm.at[pl.ds(v0 // 2, _TAILV // 2)])


@functools.partial(
    pl.kernel,
    out_type=jax.ShapeDtypeStruct((_T, _EMB, _B), jnp.float32),
    mesh=_mesh,
    scratch_types=[
        pltpu.VMEM((_T, _BBLK), jnp.int32),
        pltpu.VMEM((_BBLK,), jnp.int32),
        pltpu.VMEM((_BBLK,), jnp.int32),
        pltpu.VMEM((_BBLK,), jnp.int32),
        pltpu.VMEM((_BBLK,), jnp.int32),
        pltpu.VMEM((_BBLK, 2 * _EMB), jnp.float32),
        pltpu.VMEM((_BBLK, 2 * _EMB), jnp.float32),
        pltpu.VMEM((_EMB, _BBLK), jnp.float32),
        pltpu.VMEM((_EMB, _BBLK), jnp.float32),
        pltpu.SemaphoreType.DMA,
        pltpu.SemaphoreType.DMA,
        pltpu.SemaphoreType.DMA,
        pltpu.SemaphoreType.DMA,
    ],
    compiler_params=pltpu.CompilerParams(needs_layout_passes=False),
)
def _sc_gather(x_hbm, tbl_hbm, out_hbm, idx_all, gi0, gi1, hf0, hf1,
               gb0, gb1, ob0, ob1, g0, g1, s0, s1):
    nc = plsc.get_sparse_core_info().num_cores
    wid = lax.axis_index("s") * nc + lax.axis_index("c")
    b0 = wid * _BBLK

    gidx = (gi0, gi1)
    half = (hf0, hf1)
    gbuf = (gb0, gb1)
    obuf = (ob0, ob1)
    gsem = (g0, g1)
    ssem = (s0, s1)

    pltpu.sync_copy(x_hbm.at[:, pl.ds(b0, _BBLK)], idx_all)

    def prep_and_fire(t, s):
        for j in range(_NJ):
            iv = idx_all[t, pl.ds(j * _L, _L)]
            gidx[s][pl.ds(j * _L, _L)] = lax.shift_right_logical(iv, 1)
            half[s][pl.ds(j * _L, _L)] = lax.shift_left(
                lax.bitwise_and(iv, 1), 6)
        pltpu.async_copy(tbl_hbm.at[gidx[s]], gbuf[s], gsem[s])

    def wait_gather(s):
        pltpu.make_async_copy(tbl_hbm.at[pl.ds(0, _BBLK)], gbuf[s],
                              gsem[s]).wait()

    def transpose_block(s):
        p0 = tuple(half[s][pl.ds(j * _L, _L)] for j in range(_NJ))

        @plsc.parallel_loop(0, _EMB, unroll=4, carry=p0)
        def erow(e0, p):
            ev = lax.bitwise_and(lax.iota(jnp.int32, _L) + e0, _EMB - 1)
            for j in range(_NJ):
                rows_b = lax.iota(jnp.int32, _L) + (j * _L)
                v = plsc.load_gather(gbuf[s], [rows_b, p[j] + ev])
                plsc.store_scatter(obuf[s], [ev, rows_b], v)
            return p

    def fire_store(t, s):
        pltpu.async_copy(obuf[s], out_hbm.at[t, :, pl.ds(b0, _BBLK)],
                         ssem[s])

    def wait_store(s):
        pltpu.make_async_copy(obuf[s], out_hbm.at[0, :, pl.ds(0, _BBLK)],
                              ssem[s]).wait()

    prep_and_fire(0, 0)
    prep_and_fire(1, 1)

    for s in range(2):
        wait_gather(s)
        transpose_block(s)
        fire_store(s, s)
        prep_and_fire(s + 2, s)

    def body(i, carry):
        for s in range(2):
            t = 2 + 2 * i + s
            wait_gather(s)
            wait_store(s)
            transpose_block(s)
            fire_store(t, s)
            prep_and_fire(t + 2, s)
        return carry

    lax.fori_loop(0, (_T - 4) // 2, body, 0)

    for s in range(2):
        t = _T - 2 + s
        wait_gather(s)
        wait_store(s)
        transpose_block(s)
        fire_store(t, s)
    for s in range(2):
        wait_store(s)


def kernel(x, table):
    tt = table.T
    tp = _sc_transpose(tt)
    out_k = _sc_gather(x.T, tp)
    return out_k.transpose(2, 0, 1)


# speedup vs baseline: 1.0345x; 1.0219x over previous
import functools

import jax
import jax.numpy as jnp
from jax import lax
from jax.experimental import pallas as pl
from jax.experimental.pallas import tpu as pltpu
from jax.experimental.pallas import tpu_sc as plsc

_B, _T, _EMB = 4096, 200, 64
_VOCAB = 1000000
_NPAIR = _VOCAB // 2
_NW = 32
_BBLK = _B // _NW
_L = 16
_NJ = _BBLK // _L

_VW = 256
_NFULL = _VOCAB // _VW
_TAILV = _VOCAB - _NFULL * _VW
_CPW = _NFULL // _NW

_mesh = plsc.VectorSubcoreMesh(core_axis_name="c", subcore_axis_name="s")


@functools.partial(
    pl.kernel,
    out_type=jax.ShapeDtypeStruct((_NPAIR, 2 * _EMB), jnp.float32),
    mesh=_mesh,
    scratch_types=[
        pltpu.VMEM((_EMB, _VW), jnp.float32),
        pltpu.VMEM((_EMB, _VW), jnp.float32),
        pltpu.VMEM((_VW // 2, 2 * _EMB), jnp.float32),
        pltpu.VMEM((_VW // 2, 2 * _EMB), jnp.float32),
        pltpu.VMEM((_EMB, _TAILV), jnp.float32),
        pltpu.VMEM((_TAILV // 2, 2 * _EMB), jnp.float32),
        pltpu.SemaphoreType.DMA,
        pltpu.SemaphoreType.DMA,
        pltpu.SemaphoreType.DMA,
        pltpu.SemaphoreType.DMA,
    ],
    compiler_params=pltpu.CompilerParams(needs_layout_passes=False),
)
def _sc_transpose(tt_hbm, tp_hbm, ib0, ib1, ob0, ob1, tib, tob,
                  r0, r1, w0, w1):
    nc = plsc.get_sparse_core_info().num_cores
    wid = lax.axis_index("s") * nc + lax.axis_index("c")

    ibuf = (ib0, ib1)
    obuf = (ob0, ob1)
    rsem = (r0, r1)
    wsem = (w0, w1)

    def fire_read(cid, s):
        pltpu.async_copy(tt_hbm.at[:, pl.ds(cid * _VW, _VW)], ibuf[s],
                         rsem[s])

    def wait_read(s):
        pltpu.make_async_copy(tt_hbm.at[:, pl.ds(0, _VW)], ibuf[s],
                              rsem[s]).wait()

    def transpose_chunk(src, dst, npair):
        iot = lax.iota(jnp.int32, _L)
        for r0 in range(0, npair, _L // 2):
            cols_src = iot + (2 * r0)
            rows_dst = lax.shift_right_logical(iot, 1) + r0
            gsel = lax.shift_left(lax.bitwise_and(iot, 1), 6)

            @plsc.parallel_loop(0, _EMB, unroll=4)
            def eloop(e0):
                ev = lax.bitwise_xor(iot, e0)
                v = plsc.load_gather(src, [ev, cols_src])
                plsc.store_scatter(dst, [rows_dst, lax.bitwise_or(gsel, ev)], v)

    def fire_write(cid, s):
        pltpu.async_copy(obuf[s], tp_hbm.at[pl.ds(cid * (_VW // 2), _VW // 2)],
                         wsem[s])

    def wait_write(s):
        pltpu.make_async_copy(obuf[s], tp_hbm.at[pl.ds(0, _VW // 2)],
                              wsem[s]).wait()

    fire_read(wid, 0)
    fire_read(wid + _NW, 1)

    for s in range(2):
        wait_read(s)
        transpose_chunk(ibuf[s], obuf[s], _VW // 2)
        fire_write(wid + s * _NW, s)
        fire_read(wid + (s + 2) * _NW, s)

    def body(i, carry):
        for s in range(2):
            idx = 2 + 2 * i + s
            cid = wid + idx * _NW
            wait_read(s)
            wait_write(s)
            transpose_chunk(ibuf[s], obuf[s], _VW // 2)
            fire_write(cid, s)

            @pl.when(idx + 2 < _CPW)
            def _():
                fire_read(wid + (idx + 2) * _NW, s)
        return carry

    lax.fori_loop(0, (_CPW - 2) // 2, body, 0)
    for s in range(2):
        wait_write(s)

    @pl.when(wid < 2)
    def _():
        cid = _NFULL - 2 + wid
        pltpu.sync_copy(tt_hbm.at[:, pl.ds(cid * _VW, _VW)], ib0)
        transpose_chunk(ib0, ob0, _VW // 2)
        pltpu.sync_copy(ob0, tp_hbm.at[pl.ds(cid * (_VW // 2), _VW // 2)])

    @pl.when(wid == 2)
    def _():
        v0 = _NFULL * _VW
        pltpu.sync_copy(tt_hbm.at[:, pl.ds(v0, _TAILV)], tib)
        transpose_chunk(tib, tob, _TAILV // 2)
        pltpu.sync_copy(tob, tp_hbm.at[pl.ds(v0 // 2, _TAILV // 2)])


@functools.partial(
    pl.kernel,
    out_type=jax.ShapeDtypeStruct((_T, _EMB, _B), jnp.float32),
    mesh=_mesh,
    scratch_types=[
        pltpu.VMEM((_T, _BBLK), jnp.int32),
        pltpu.VMEM((_BBLK,), jnp.int32),
        pltpu.VMEM((_BBLK,), jnp.int32),
        pltpu.VMEM((_BBLK,), jnp.int32),
        pltpu.VMEM((_BBLK,), jnp.int32),
        pltpu.VMEM((_BBLK, 2 * _EMB), jnp.float32),
        pltpu.VMEM((_BBLK, 2 * _EMB), jnp.float32),
        pltpu.VMEM((_EMB, _BBLK), jnp.float32),
        pltpu.VMEM((_EMB, _BBLK), jnp.float32),
        pltpu.SemaphoreType.DMA,
        pltpu.SemaphoreType.DMA,
        pltpu.SemaphoreType.DMA,
        pltpu.SemaphoreType.DMA,
    ],
    compiler_params=pltpu.CompilerParams(needs_layout_passes=False),
)
def _sc_gather(x_hbm, tbl_hbm, out_hbm, idx_all, gi0, gi1, hf0, hf1,
               gb0, gb1, ob0, ob1, g0, g1, s0, s1):
    nc = plsc.get_sparse_core_info().num_cores
    wid = lax.axis_index("s") * nc + lax.axis_index("c")
    b0 = wid * _BBLK

    gidx = (gi0, gi1)
    half = (hf0, hf1)
    gbuf = (gb0, gb1)
    obuf = (ob0, ob1)
    gsem = (g0, g1)
    ssem = (s0, s1)

    pltpu.sync_copy(x_hbm.at[:, pl.ds(b0, _BBLK)], idx_all)

    def prep_and_fire(t, s):
        for j in range(_NJ):
            iv = idx_all[t, pl.ds(j * _L, _L)]
            gidx[s][pl.ds(j * _L, _L)] = lax.shift_right_logical(iv, 1)
            half[s][pl.ds(j * _L, _L)] = lax.shift_left(
                lax.bitwise_and(iv, 1), 6)
        pltpu.async_copy(tbl_hbm.at[gidx[s]], gbuf[s], gsem[s])

    def wait_gather(s):
        pltpu.make_async_copy(tbl_hbm.at[pl.ds(0, _BBLK)], gbuf[s],
                              gsem[s]).wait()

    def transpose_block(s):
        p0 = tuple(half[s][pl.ds(j * _L, _L)] for j in range(_NJ))

        @plsc.parallel_loop(0, _EMB, unroll=4, carry=p0)
        def erow(e0, p):
            ev = lax.bitwise_xor(lax.iota(jnp.int32, _L), e0)
            for j in range(_NJ):
                rows_b = lax.iota(jnp.int32, _L) + (j * _L)
                v = plsc.load_gather(gbuf[s], [rows_b, p[j] + ev])
                plsc.store_scatter(obuf[s], [ev, rows_b], v)
            return p

    def fire_store(t, s):
        pltpu.async_copy(obuf[s], out_hbm.at[t, :, pl.ds(b0, _BBLK)],
                         ssem[s])

    def wait_store(s):
        pltpu.make_async_copy(obuf[s], out_hbm.at[0, :, pl.ds(0, _BBLK)],
                              ssem[s]).wait()

    prep_and_fire(0, 0)
    prep_and_fire(1, 1)

    for s in range(2):
        wait_gather(s)
        transpose_block(s)
        fire_store(s, s)
        prep_and_fire(s + 2, s)

    def body(i, carry):
        for s in range(2):
            t = 2 + 2 * i + s
            wait_gather(s)
            wait_store(s)
            transpose_block(s)
            fire_store(t, s)
            prep_and_fire(t + 2, s)
        return carry

    lax.fori_loop(0, (_T - 4) // 2, body, 0)

    for s in range(2):
        t = _T - 2 + s
        wait_gather(s)
        wait_store(s)
        transpose_block(s)
        fire_store(t, s)
    for s in range(2):
        wait_store(s)


def kernel(x, table):
    tt = table.T
    tp = _sc_transpose(tt)
    out_k = _sc_gather(x.T, tp)
    return out_k.transpose(2, 0, 1)
